# baseline (device time: 183459 ns/iter reference)
import jax
import jax.numpy as jnp
from jax import lax
from jax.experimental import pallas as pl
from jax.experimental.pallas import tpu as pltpu

N_DEV = 8


def kernel(x, w_mat):
    m, k_loc = x.shape
    n = w_mat.shape[1]
    chunk = m // N_DEV

    def body(x_ref, w_ref, out_ref, send_ref, comm_ref, send_sems, recv_sems):
        my = lax.axis_index("i")
        left = (my - 1) % N_DEV
        right = (my + 1) % N_DEV

        barrier_sem = pltpu.get_barrier_semaphore()
        for nbr in [left, right]:
            pl.semaphore_signal(
                barrier_sem, inc=1,
                device_id=(nbr,), device_id_type=pl.DeviceIdType.MESH,
            )
        pl.semaphore_wait(barrier_sem, 2)

        def partial_for(c):
            return jnp.dot(
                x_ref[pl.ds(c * chunk, chunk), :], w_ref[...],
                preferred_element_type=jnp.float32,
            )

        for s in range(N_DEV - 1):
            c = (my - 1 - s) % N_DEV
            part = partial_for(c)
            if s == 0:
                acc = part
            else:
                acc = part + comm_ref[s - 1]
            send_ref[s % 2] = acc
            rdma = pltpu.make_async_remote_copy(
                src_ref=send_ref.at[s % 2],
                dst_ref=comm_ref.at[s],
                send_sem=send_sems.at[s % 2],
                recv_sem=recv_sems.at[s],
                device_id=(right,),
                device_id_type=pl.DeviceIdType.MESH,
            )
            rdma.start()
            rdma.wait()

        y = partial_for(my) + comm_ref[N_DEV - 2]
        out_ref[...] = y * jax.nn.sigmoid(y)

        def _exit(second_barrier):
            for nbr in [left, right]:
                pl.semaphore_signal(
                    second_barrier, inc=1,
                    device_id=(nbr,), device_id_type=pl.DeviceIdType.MESH,
                )
            pl.semaphore_wait(second_barrier, 2)

        pl.run_scoped(_exit, second_barrier=pltpu.SemaphoreType.REGULAR)

    return pl.pallas_call(
        body,
        out_shape=jax.ShapeDtypeStruct((chunk, n), jnp.float32),
        in_specs=[
            pl.BlockSpec(memory_space=pltpu.VMEM),
            pl.BlockSpec(memory_space=pltpu.VMEM),
        ],
        out_specs=pl.BlockSpec(memory_space=pltpu.VMEM),
        scratch_shapes=[
            pltpu.VMEM((2, chunk, n), jnp.float32),
            pltpu.VMEM((N_DEV - 1, chunk, n), jnp.float32),
            pltpu.SemaphoreType.DMA((2,)),
            pltpu.SemaphoreType.DMA((N_DEV - 1,)),
        ],
        compiler_params=pltpu.CompilerParams(collective_id=0),
    )(x, w_mat)


# device time: 108019 ns/iter; 1.6984x vs baseline; 1.6984x over previous
import jax
import jax.numpy as jnp
from jax import lax
from jax.experimental import pallas as pl
from jax.experimental.pallas import tpu as pltpu

N_DEV = 8


def kernel(x, w_mat):
    m, k_loc = x.shape
    n = w_mat.shape[1]
    chunk = m // N_DEV
    half = n // 2

    def body(x_ref, w_ref, out_ref,
             send_r, send_l, comm_r, comm_l,
             send_sems_r, send_sems_l, recv_sems_r, recv_sems_l):
        my = lax.axis_index("i")
        left = (my - 1) % N_DEV
        right = (my + 1) % N_DEV

        barrier_sem = pltpu.get_barrier_semaphore()
        for nbr in [left, right]:
            pl.semaphore_signal(
                barrier_sem, inc=1,
                device_id=(nbr,), device_id_type=pl.DeviceIdType.MESH,
            )
        pl.semaphore_wait(barrier_sem, 2)

        def part_r(c):
            return jnp.dot(
                x_ref[pl.ds(c * chunk, chunk), :], w_ref[:, :half],
                preferred_element_type=jnp.float32,
            )

        def part_l(c):
            return jnp.dot(
                x_ref[pl.ds(c * chunk, chunk), :], w_ref[:, half:],
                preferred_element_type=jnp.float32,
            )

        def make_rdma(direction, s):
            if direction == "r":
                return pltpu.make_async_remote_copy(
                    src_ref=send_r.at[s % 2],
                    dst_ref=comm_r.at[s],
                    send_sem=send_sems_r.at[s % 2],
                    recv_sem=recv_sems_r.at[s],
                    device_id=(right,),
                    device_id_type=pl.DeviceIdType.MESH,
                )
            return pltpu.make_async_remote_copy(
                src_ref=send_l.at[s % 2],
                dst_ref=comm_l.at[s],
                send_sem=send_sems_l.at[s % 2],
                recv_sem=recv_sems_l.at[s],
                device_id=(left,),
                device_id_type=pl.DeviceIdType.MESH,
            )

        rdmas_r = [None] * (N_DEV - 1)
        rdmas_l = [None] * (N_DEV - 1)
        send_r[0] = part_r((my - 1) % N_DEV)
        send_l[0] = part_l((my + 1) % N_DEV)
        rdmas_r[0] = make_rdma("r", 0)
        rdmas_l[0] = make_rdma("l", 0)
        rdmas_r[0].start()
        rdmas_l[0].start()

        for s in range(1, N_DEV - 1):
            pr = part_r((my - 1 - s) % N_DEV)
            pl_ = part_l((my + 1 + s) % N_DEV)
            rdmas_r[s - 1].wait_recv()
            rdmas_l[s - 1].wait_recv()
            if s >= 2:
                rdmas_r[s - 2].wait_send()
                rdmas_l[s - 2].wait_send()
            send_r[s % 2] = pr + comm_r[s - 1]
            send_l[s % 2] = pl_ + comm_l[s - 1]
            rdmas_r[s] = make_rdma("r", s)
            rdmas_l[s] = make_rdma("l", s)
            rdmas_r[s].start()
            rdmas_l[s].start()

        own_r = part_r(my)
        own_l = part_l(my)
        rdmas_r[N_DEV - 2].wait_recv()
        rdmas_l[N_DEV - 2].wait_recv()
        y_r = own_r + comm_r[N_DEV - 2]
        y_l = own_l + comm_l[N_DEV - 2]
        out_ref[:, :half] = y_r * jax.nn.sigmoid(y_r)
        out_ref[:, half:] = y_l * jax.nn.sigmoid(y_l)

        for s in (N_DEV - 3, N_DEV - 2):
            rdmas_r[s].wait_send()
            rdmas_l[s].wait_send()

        def _exit(second_barrier):
            for nbr in [left, right]:
                pl.semaphore_signal(
                    second_barrier, inc=1,
                    device_id=(nbr,), device_id_type=pl.DeviceIdType.MESH,
                )
            pl.semaphore_wait(second_barrier, 2)

        pl.run_scoped(_exit, second_barrier=pltpu.SemaphoreType.REGULAR)

    return pl.pallas_call(
        body,
        out_shape=jax.ShapeDtypeStruct((chunk, n), jnp.float32),
        in_specs=[
            pl.BlockSpec(memory_space=pltpu.VMEM),
            pl.BlockSpec(memory_space=pltpu.VMEM),
        ],
        out_specs=pl.BlockSpec(memory_space=pltpu.VMEM),
        scratch_shapes=[
            pltpu.VMEM((2, chunk, half), jnp.float32),
            pltpu.VMEM((2, chunk, half), jnp.float32),
            pltpu.VMEM((N_DEV - 1, chunk, half), jnp.float32),
            pltpu.VMEM((N_DEV - 1, chunk, half), jnp.float32),
            pltpu.SemaphoreType.DMA((2,)),
            pltpu.SemaphoreType.DMA((2,)),
            pltpu.SemaphoreType.DMA((N_DEV - 1,)),
            pltpu.SemaphoreType.DMA((N_DEV - 1,)),
        ],
        compiler_params=pltpu.CompilerParams(collective_id=0),
    )(x, w_mat)


# device time: 91371 ns/iter; 2.0078x vs baseline; 1.1822x over previous
import jax
import jax.numpy as jnp
from jax import lax
from jax.experimental import pallas as pl
from jax.experimental.pallas import tpu as pltpu

N_DEV = 8
N_SUB = 2


def kernel(x, w_mat):
    m, k_loc = x.shape
    n = w_mat.shape[1]
    chunk = m // N_DEV
    half = n // 2
    subw = half // N_SUB
    n_streams = 2 * N_SUB

    def body(x_ref, w_ref, out_ref,
             send_buf, comm_buf, send_sems, recv_sems):
        my = lax.axis_index("i")
        left = (my - 1) % N_DEV
        right = (my + 1) % N_DEV

        barrier_sem = pltpu.get_barrier_semaphore()
        for nbr in [left, right]:
            pl.semaphore_signal(
                barrier_sem, inc=1,
                device_id=(nbr,), device_id_type=pl.DeviceIdType.MESH,
            )
        pl.semaphore_wait(barrier_sem, 2)

        def part(c, col0, ncol):
            return jnp.dot(
                x_ref[pl.ds(c * chunk, chunk), :], w_ref[:, col0:col0 + ncol],
                preferred_element_type=jnp.float32,
            )

        def make_rdma(k, s, target):
            return pltpu.make_async_remote_copy(
                src_ref=send_buf.at[k, s % 2],
                dst_ref=comm_buf.at[k, s],
                send_sem=send_sems.at[k, s % 2],
                recv_sem=recv_sems.at[k, s],
                device_id=(target,),
                device_id_type=pl.DeviceIdType.MESH,
            )

        rdmas = [[None] * (N_DEV - 1) for _ in range(n_streams)]

        def hop(s):
            pr = part((my - 1 - s) % N_DEV, 0, half)
            pl_ = part((my + 1 + s) % N_DEV, half, half)
            for j in range(N_SUB):
                for d, pmat, tgt in ((0, pr, right), (1, pl_, left)):
                    k = d * N_SUB + j
                    stripe = pmat[:, j * subw:(j + 1) * subw]
                    if s > 0:
                        rdmas[k][s - 1].wait_recv()
                        stripe = stripe + comm_buf[k, s - 1]
                    if s >= 2:
                        rdmas[k][s - 2].wait_send()
                    send_buf[k, s % 2] = stripe
                    rdmas[k][s] = make_rdma(k, s, tgt)
                    rdmas[k][s].start()

        for s in range(N_DEV - 1):
            hop(s)

        own = part(my, 0, n)
        for j in range(N_SUB):
            for d in (0, 1):
                k = d * N_SUB + j
                col0 = d * half + j * subw
                rdmas[k][N_DEV - 2].wait_recv()
                y = own[:, col0:col0 + subw] + comm_buf[k, N_DEV - 2]
                out_ref[:, col0:col0 + subw] = y * jax.nn.sigmoid(y)

        for s in (N_DEV - 3, N_DEV - 2):
            for k in range(n_streams):
                rdmas[k][s].wait_send()

        def _exit(second_barrier):
            for nbr in [left, right]:
                pl.semaphore_signal(
                    second_barrier, inc=1,
                    device_id=(nbr,), device_id_type=pl.DeviceIdType.MESH,
                )
            pl.semaphore_wait(second_barrier, 2)

        pl.run_scoped(_exit, second_barrier=pltpu.SemaphoreType.REGULAR)

    return pl.pallas_call(
        body,
        out_shape=jax.ShapeDtypeStruct((chunk, n), jnp.float32),
        in_specs=[
            pl.BlockSpec(memory_space=pltpu.VMEM),
            pl.BlockSpec(memory_space=pltpu.VMEM),
        ],
        out_specs=pl.BlockSpec(memory_space=pltpu.VMEM),
        scratch_shapes=[
            pltpu.VMEM((n_streams, 2, chunk, subw), jnp.float32),
            pltpu.VMEM((n_streams, N_DEV - 1, chunk, subw), jnp.float32),
            pltpu.SemaphoreType.DMA((n_streams, 2)),
            pltpu.SemaphoreType.DMA((n_streams, N_DEV - 1)),
        ],
        compiler_params=pltpu.CompilerParams(collective_id=0),
    )(x, w_mat)
